# baseline (device time: 75566 ns/iter reference)
import jax
import jax.numpy as jnp
from jax import lax
from jax.experimental import pallas as pl
from jax.experimental.pallas import tpu as pltpu

N_DEV = 16
CAP = 128
PAD_L = 384


def kernel(x, router_W, route_idx, expert_W, shared_W):
    T, D = x.shape
    E_LOC, _, H = expert_W.shape
    E_TOT = router_W.shape[1]
    f32 = jnp.float32
    bf16 = jnp.bfloat16

    def body(x_ref, rW_ref, idx_ref, eW_ref, sW_ref, out_ref,
             pack_ref, disp_ref, yout_ref, yin_ref,
             dsend, drecv, ysend, yrecv):
        my = lax.axis_index("i")

        barrier_sem = pltpu.get_barrier_semaphore()
        for k in range(1, N_DEV):
            peer = lax.rem(my + k, N_DEV)
            pl.semaphore_signal(barrier_sem, inc=1, device_id=(peer,),
                                device_id_type=pl.DeviceIdType.MESH)
        pl.semaphore_wait(barrier_sem, N_DEV - 1)

        xv = x_ref[:, :]
        scores = jnp.dot(xv, rW_ref[:, :], preferred_element_type=f32,
                         precision=lax.Precision.HIGHEST)
        mx = jnp.max(scores, axis=-1, keepdims=True)
        ex = jnp.exp(scores - mx)
        probs = ex / jnp.sum(ex, axis=-1, keepdims=True)
        eidx = idx_ref[:, :]
        oh_tok = lax.broadcasted_iota(jnp.int32, (T, E_TOT), 1) == eidx
        gp = jnp.sum(jnp.where(oh_tok, probs, 0.0), axis=-1,
                     keepdims=True)
        xg = xv * gp

        dst_tok = eidx // E_LOC

        iota16 = lax.broadcasted_iota(jnp.int32, (T, N_DEV), 1)
        Mf = jnp.where(iota16 == dst_tok, 1.0, 0.0)
        ir = lax.broadcasted_iota(jnp.int32, (T, T), 0)
        ic = lax.broadcasted_iota(jnp.int32, (T, T), 1)
        tri = jnp.where(ic <= ir, 1.0, 0.0)
        prefixs = jnp.dot(tri, Mf, preferred_element_type=f32)

        iota_cap = lax.broadcasted_iota(jnp.int32, (T, CAP), 1)

        def make_Pt(d):
            mask_d = jnp.where(dst_tok == d, 1.0, 0.0)
            pre_d = jnp.sum(jnp.where(iota16 == d, prefixs, 0.0),
                            axis=-1, keepdims=True)
            pre_i = pre_d.astype(jnp.int32)
            return jnp.where(iota_cap == pre_i - 1, mask_d, 0.0)

        near_first = sorted(range(1, N_DEV), key=lambda k: min(k, N_DEV - k))
        far_first = near_first[::-1]

        PtAll = jnp.concatenate(
            [make_Pt(lax.rem(my + k, N_DEV)) for k in range(1, N_DEV)],
            axis=1)
        xpAll = lax.dot_general(PtAll, xg, (((0,), (0,)), ((), ())),
                                preferred_element_type=f32)
        jloc_oh = jnp.where(
            lax.broadcasted_iota(jnp.int32, (T, E_LOC), 1) == eidx % E_LOC,
            1.0, 0.0)
        JAll = lax.dot_general(PtAll, jloc_oh, (((0,), (0,)), ((), ())),
                               preferred_element_type=f32)
        payload = jnp.concatenate(
            [xpAll, JAll, jnp.zeros(((N_DEV - 1) * CAP, PAD_L - D - E_LOC),
                                    f32)],
            axis=1).astype(bf16)
        disp_rdmas = []
        for k in far_first:
            pack_ref[k - 1] = payload[(k - 1) * CAP:k * CAP]
            rdma = pltpu.make_async_remote_copy(
                src_ref=pack_ref.at[k - 1],
                dst_ref=disp_ref.at[k - 1],
                send_sem=dsend.at[k - 1],
                recv_sem=drecv.at[k - 1],
                device_id=(lax.rem(my + k, N_DEV),),
                device_id_type=pl.DeviceIdType.MESH,
            )
            rdma.start()
            disp_rdmas.append(rdma)

        acc = jnp.dot(xv, sW_ref[:, :], preferred_element_type=f32)
        for j in range(E_LOC):
            gid = my * E_LOC + j
            cj = jnp.where(eidx == gid, gp, 0.0)
            acc = acc + jnp.dot(xv * cj, eW_ref[j],
                                preferred_element_type=f32)

        y_rdmas = []
        for grp in (near_first[:8], near_first[8:]):
            for k in grp:
                src = lax.rem(my + (N_DEV - k), N_DEV)
                recv = pltpu.make_async_remote_copy(
                    src_ref=pack_ref.at[k - 1],
                    dst_ref=disp_ref.at[k - 1],
                    send_sem=dsend.at[k - 1],
                    recv_sem=drecv.at[k - 1],
                    device_id=(src,),
                    device_id_type=pl.DeviceIdType.MESH,
                )
                recv.wait_recv()
            prs = jnp.concatenate([disp_ref[k - 1] for k in grp],
                                  axis=0)
            xr = prs[:, :D].astype(f32)
            y = jnp.zeros((len(grp) * CAP, H), f32)
            for j in range(E_LOC):
                cj = prs[:, D + j:D + j + 1].astype(f32)
                y = y + jnp.dot(xr * cj, eW_ref[j],
                                preferred_element_type=f32)
            yb = y.astype(bf16)
            for i, k in enumerate(grp):
                yout_ref[k - 1] = yb[i * CAP:(i + 1) * CAP]
                rdma = pltpu.make_async_remote_copy(
                    src_ref=yout_ref.at[k - 1],
                    dst_ref=yin_ref.at[k - 1],
                    send_sem=ysend.at[k - 1],
                    recv_sem=yrecv.at[k - 1],
                    device_id=(lax.rem(my + (N_DEV - k), N_DEV),),
                    device_id_type=pl.DeviceIdType.MESH,
                )
                rdma.start()
                y_rdmas.append(rdma)

        for k in near_first:
            recv = pltpu.make_async_remote_copy(
                src_ref=yout_ref.at[k - 1],
                dst_ref=yin_ref.at[k - 1],
                send_sem=ysend.at[k - 1],
                recv_sem=yrecv.at[k - 1],
                device_id=(lax.rem(my + k, N_DEV),),
                device_id_type=pl.DeviceIdType.MESH,
            )
            recv.wait_recv()
        ysAll = jnp.concatenate(
            [yin_ref[k - 1].astype(f32) for k in range(1, N_DEV)],
            axis=0)
        acc = acc + jnp.dot(PtAll, ysAll, preferred_element_type=f32)

        out_ref[:, :] = acc

        for r in disp_rdmas:
            r.wait_send()
        for r in y_rdmas:
            r.wait_send()

    return pl.pallas_call(
        body,
        out_shape=jax.ShapeDtypeStruct((T, H), jnp.float32),
        in_specs=[pl.BlockSpec(memory_space=pltpu.VMEM)] * 5,
        out_specs=pl.BlockSpec(memory_space=pltpu.VMEM),
        scratch_shapes=[
            pltpu.VMEM((N_DEV - 1, CAP, PAD_L), jnp.bfloat16),
            pltpu.VMEM((N_DEV - 1, CAP, PAD_L), jnp.bfloat16),
            pltpu.VMEM((N_DEV - 1, CAP, H), jnp.bfloat16),
            pltpu.VMEM((N_DEV - 1, CAP, H), jnp.bfloat16),
            pltpu.SemaphoreType.DMA((N_DEV - 1,)),
            pltpu.SemaphoreType.DMA((N_DEV - 1,)),
            pltpu.SemaphoreType.DMA((N_DEV - 1,)),
            pltpu.SemaphoreType.DMA((N_DEV - 1,)),
        ],
        compiler_params=pltpu.CompilerParams(
            collective_id=0,
            vmem_limit_bytes=96 * 1024 * 1024,
        ),
    )(x, router_W, route_idx, expert_W, shared_W)


# device time: 73840 ns/iter; 1.0234x vs baseline; 1.0234x over previous
import jax
import jax.numpy as jnp
from jax import lax
from jax.experimental import pallas as pl
from jax.experimental.pallas import tpu as pltpu

N_DEV = 16
CAP = 128
PAD_L = 384


def kernel(x, router_W, route_idx, expert_W, shared_W):
    T, D = x.shape
    E_LOC, _, H = expert_W.shape
    E_TOT = router_W.shape[1]
    f32 = jnp.float32
    bf16 = jnp.bfloat16

    def body(x_ref, rW_ref, idx_ref, eW_ref, sW_ref, out_ref,
             pack_ref, disp_ref, yout_ref, yin_ref,
             dsend, drecv, ysend, yrecv):
        my = lax.axis_index("i")

        barrier_sem = pltpu.get_barrier_semaphore()
        for k in range(1, N_DEV):
            peer = lax.rem(my + k, N_DEV)
            pl.semaphore_signal(barrier_sem, inc=1, device_id=(peer,),
                                device_id_type=pl.DeviceIdType.MESH)
        pl.semaphore_wait(barrier_sem, N_DEV - 1)

        xv = x_ref[:, :]
        scores = jnp.dot(xv, rW_ref[:, :], preferred_element_type=f32)
        mx = jnp.max(scores, axis=-1, keepdims=True)
        ex = jnp.exp(scores - mx)
        probs = ex / jnp.sum(ex, axis=-1, keepdims=True)
        eidx = idx_ref[:, :]
        oh_tok = lax.broadcasted_iota(jnp.int32, (T, E_TOT), 1) == eidx
        gp = jnp.sum(jnp.where(oh_tok, probs, 0.0), axis=-1,
                     keepdims=True)
        xg = xv * gp

        dst_tok = eidx // E_LOC

        iota16 = lax.broadcasted_iota(jnp.int32, (T, N_DEV), 1)
        Mf = jnp.where(iota16 == dst_tok, 1.0, 0.0)
        ir = lax.broadcasted_iota(jnp.int32, (T, T), 0)
        ic = lax.broadcasted_iota(jnp.int32, (T, T), 1)
        tri = jnp.where(ic <= ir, 1.0, 0.0)
        prefixs = jnp.dot(tri, Mf, preferred_element_type=f32)

        iota_cap = lax.broadcasted_iota(jnp.int32, (T, CAP), 1)

        def make_Pt(d):
            mask_d = jnp.where(dst_tok == d, 1.0, 0.0)
            pre_d = jnp.sum(jnp.where(iota16 == d, prefixs, 0.0),
                            axis=-1, keepdims=True)
            pre_i = pre_d.astype(jnp.int32)
            return jnp.where(iota_cap == pre_i - 1, mask_d, 0.0)

        near_first = sorted(range(1, N_DEV), key=lambda k: min(k, N_DEV - k))
        far_first = near_first[::-1]

        PtAll = jnp.concatenate(
            [make_Pt(lax.rem(my + k, N_DEV)) for k in range(1, N_DEV)],
            axis=1)
        xpAll = lax.dot_general(PtAll, xg, (((0,), (0,)), ((), ())),
                                preferred_element_type=f32)
        jloc_oh = jnp.where(
            lax.broadcasted_iota(jnp.int32, (T, E_LOC), 1) == eidx % E_LOC,
            1.0, 0.0)
        JAll = lax.dot_general(PtAll, jloc_oh, (((0,), (0,)), ((), ())),
                               preferred_element_type=f32)
        payload = jnp.concatenate(
            [xpAll, JAll, jnp.zeros(((N_DEV - 1) * CAP, PAD_L - D - E_LOC),
                                    f32)],
            axis=1).astype(bf16)
        disp_rdmas = []
        for k in far_first:
            pack_ref[k - 1] = payload[(k - 1) * CAP:k * CAP]
            rdma = pltpu.make_async_remote_copy(
                src_ref=pack_ref.at[k - 1],
                dst_ref=disp_ref.at[k - 1],
                send_sem=dsend.at[k - 1],
                recv_sem=drecv.at[k - 1],
                device_id=(lax.rem(my + k, N_DEV),),
                device_id_type=pl.DeviceIdType.MESH,
            )
            rdma.start()
            disp_rdmas.append(rdma)

        acc = jnp.dot(xv, sW_ref[:, :], preferred_element_type=f32)
        for j in range(E_LOC):
            gid = my * E_LOC + j
            cj = jnp.where(eidx == gid, gp, 0.0)
            acc = acc + jnp.dot(xv * cj, eW_ref[j],
                                preferred_element_type=f32)

        y_rdmas = []
        for k in near_first:
            src = lax.rem(my + (N_DEV - k), N_DEV)
            recv = pltpu.make_async_remote_copy(
                src_ref=pack_ref.at[k - 1],
                dst_ref=disp_ref.at[k - 1],
                send_sem=dsend.at[k - 1],
                recv_sem=drecv.at[k - 1],
                device_id=(src,),
                device_id_type=pl.DeviceIdType.MESH,
            )
            recv.wait_recv()
            pr = disp_ref[k - 1]
            xr = pr[:, :D].astype(f32)
            y = jnp.zeros((CAP, H), f32)
            for j in range(E_LOC):
                cj = pr[:, D + j:D + j + 1].astype(f32)
                y = y + jnp.dot(xr * cj, eW_ref[j],
                                preferred_element_type=f32)
            yout_ref[k - 1] = y.astype(bf16)
            rdma = pltpu.make_async_remote_copy(
                src_ref=yout_ref.at[k - 1],
                dst_ref=yin_ref.at[k - 1],
                send_sem=ysend.at[k - 1],
                recv_sem=yrecv.at[k - 1],
                device_id=(src,),
                device_id_type=pl.DeviceIdType.MESH,
            )
            rdma.start()
            y_rdmas.append(rdma)

        for k in near_first:
            recv = pltpu.make_async_remote_copy(
                src_ref=yout_ref.at[k - 1],
                dst_ref=yin_ref.at[k - 1],
                send_sem=ysend.at[k - 1],
                recv_sem=yrecv.at[k - 1],
                device_id=(lax.rem(my + k, N_DEV),),
                device_id_type=pl.DeviceIdType.MESH,
            )
            recv.wait_recv()
        ysAll = jnp.concatenate(
            [yin_ref[k - 1].astype(f32) for k in range(1, N_DEV)],
            axis=0)
        acc = acc + jnp.dot(PtAll, ysAll, preferred_element_type=f32)

        out_ref[:, :] = acc

        for r in disp_rdmas:
            r.wait_send()
        for r in y_rdmas:
            r.wait_send()

    return pl.pallas_call(
        body,
        out_shape=jax.ShapeDtypeStruct((T, H), jnp.float32),
        in_specs=[pl.BlockSpec(memory_space=pltpu.VMEM)] * 5,
        out_specs=pl.BlockSpec(memory_space=pltpu.VMEM),
        scratch_shapes=[
            pltpu.VMEM((N_DEV - 1, CAP, PAD_L), jnp.bfloat16),
            pltpu.VMEM((N_DEV - 1, CAP, PAD_L), jnp.bfloat16),
            pltpu.VMEM((N_DEV - 1, CAP, H), jnp.bfloat16),
            pltpu.VMEM((N_DEV - 1, CAP, H), jnp.bfloat16),
            pltpu.SemaphoreType.DMA((N_DEV - 1,)),
            pltpu.SemaphoreType.DMA((N_DEV - 1,)),
            pltpu.SemaphoreType.DMA((N_DEV - 1,)),
            pltpu.SemaphoreType.DMA((N_DEV - 1,)),
        ],
        compiler_params=pltpu.CompilerParams(
            collective_id=0,
            vmem_limit_bytes=96 * 1024 * 1024,
        ),
    )(x, router_W, route_idx, expert_W, shared_W)
